# Initial kernel scaffold; baseline (speedup 1.0000x reference)
import jax
import jax.numpy as jnp
from jax.experimental import pallas as pl


def kernel(x, edge_index, W1, b1, W2, b2, W3, b3, W_out, b_out):
    raise NotImplementedError("write your pallas kernel here")


# speedup vs baseline: 15.6165x; 15.6165x over previous
import functools

import jax
import jax.numpy as jnp
from jax import lax
from jax.experimental import pallas as pl
from jax.experimental.pallas import tpu as pltpu
from jax.experimental.pallas import tpu_sc as plsc

N = 10000
E = 320000
D = 128
D_OUT = 64

NC, NS = 2, 16
NW = NC * NS
EPW = E // NW
K = 80
NCHNK = EPW // K
NP = 10240
RPT = NP // NS

_mesh = plsc.VectorSubcoreMesh(core_axis_name="c", subcore_axis_name="s")


@functools.partial(
    pl.kernel,
    out_type=jax.ShapeDtypeStruct((NC, NP, D), jnp.float32),
    mesh=_mesh,
    scratch_types=[
        pltpu.VMEM((NCHNK, K), jnp.int32),
        pltpu.VMEM((K, D), jnp.float32),
        pltpu.VMEM_SHARED((NP, D), jnp.float32),
    ],
)
def _sc_deg(dst_hbm, ones_hbm, z128_hbm, out_hbm, dst_v, ones_v, hist):
    c = lax.axis_index("c")
    s = lax.axis_index("s")
    wid = s * NC + c
    pltpu.sync_copy(dst_hbm.at[wid], dst_v)
    pltpu.sync_copy(ones_hbm, ones_v)
    pltpu.sync_copy(z128_hbm, hist.at[pl.ds(s * RPT, RPT)])
    plsc.subcore_barrier()

    def chunk(i, carry):
        pltpu.sync_copy(ones_v, hist.at[dst_v.at[i]], add=True)
        return carry

    lax.fori_loop(0, NCHNK, chunk, 0)
    plsc.subcore_barrier()
    pltpu.sync_copy(hist.at[pl.ds(s * RPT, RPT)],
                    out_hbm.at[c, pl.ds(s * RPT, RPT)])


@functools.partial(
    pl.kernel,
    out_type=jax.ShapeDtypeStruct((NC, NP, D), jnp.float32),
    mesh=_mesh,
    scratch_types=[
        pltpu.VMEM((NCHNK, K), jnp.int32),
        pltpu.VMEM((NCHNK, K), jnp.int32),
        pltpu.VMEM((K, D), jnp.float32),
        pltpu.VMEM_SHARED((NP, D), jnp.float32),
    ],
)
def _sc_agg(g_hbm, src_hbm, dst_hbm, z128_hbm, out_hbm,
            src_v, dst_v, rows_v, acc):
    c = lax.axis_index("c")
    s = lax.axis_index("s")
    wid = s * NC + c
    pltpu.sync_copy(src_hbm.at[wid], src_v)
    pltpu.sync_copy(dst_hbm.at[wid], dst_v)
    rows0 = s * RPT

    @pl.when(c == 0)
    def _init_self_loop():
        pltpu.sync_copy(g_hbm.at[pl.ds(rows0, RPT)], acc.at[pl.ds(rows0, RPT)])

    @pl.when(c == 1)
    def _init_zero():
        pltpu.sync_copy(z128_hbm, acc.at[pl.ds(rows0, RPT)])

    plsc.subcore_barrier()

    def chunk(i, carry):
        pltpu.sync_copy(g_hbm.at[src_v.at[i]], rows_v)
        pltpu.sync_copy(rows_v, acc.at[dst_v.at[i]], add=True)
        return carry

    lax.fori_loop(0, NCHNK, chunk, 0)
    plsc.subcore_barrier()
    pltpu.sync_copy(acc.at[pl.ds(rows0, RPT)],
                    out_hbm.at[c, pl.ds(rows0, RPT)])


R = 1000
GRID = N // R


def _tc_first_body(x_ref, w_ref, hist_ref, g_ref, dinv_ref):
    deg = hist_ref[0, :, 0:1] + hist_ref[1, :, 0:1] + 1.0
    dinv = lax.rsqrt(deg)
    h = jnp.dot(x_ref[...], w_ref[...], preferred_element_type=jnp.float32)
    g_ref[...] = h * dinv
    dinv_ref[...] = dinv


def _tc_first(x, W1, hist):
    return pl.pallas_call(
        _tc_first_body,
        grid=(GRID,),
        in_specs=[
            pl.BlockSpec((R, D), lambda i: (i, 0)),
            pl.BlockSpec((D, D), lambda i: (0, 0)),
            pl.BlockSpec((NC, R, D), lambda i: (0, i, 0)),
        ],
        out_specs=[
            pl.BlockSpec((R, D), lambda i: (i, 0)),
            pl.BlockSpec((R, 1), lambda i: (i, 0)),
        ],
        out_shape=[
            jax.ShapeDtypeStruct((NP, D), jnp.float32),
            jax.ShapeDtypeStruct((N, 1), jnp.float32),
        ],
    )(x, W1, hist)


def _tc_mid_body(p_ref, dinv_ref, b_ref, w_ref, g_ref):
    dinv = dinv_ref[...]
    agg = p_ref[0] + p_ref[1]
    xa = jnp.maximum(dinv * agg + b_ref[...], 0.0)
    h = jnp.dot(xa, w_ref[...], preferred_element_type=jnp.float32)
    g_ref[...] = h * dinv


def _tc_mid(p, dinv, b, W):
    return pl.pallas_call(
        _tc_mid_body,
        grid=(GRID,),
        in_specs=[
            pl.BlockSpec((NC, R, D), lambda i: (0, i, 0)),
            pl.BlockSpec((R, 1), lambda i: (i, 0)),
            pl.BlockSpec((1, D), lambda i: (0, 0)),
            pl.BlockSpec((D, D), lambda i: (0, 0)),
        ],
        out_specs=pl.BlockSpec((R, D), lambda i: (i, 0)),
        out_shape=jax.ShapeDtypeStruct((NP, D), jnp.float32),
    )(p, dinv, b, W)


def _tc_final_body(p_ref, dinv_ref, b3_ref, wout_ref, bout_ref, y_ref):
    x4 = dinv_ref[...] * (p_ref[0] + p_ref[1]) + b3_ref[...]
    y_ref[...] = (jnp.dot(x4, wout_ref[...], preferred_element_type=jnp.float32)
                  + bout_ref[...])


def _tc_final(p, dinv, b3, W_out, b_out):
    return pl.pallas_call(
        _tc_final_body,
        grid=(GRID,),
        in_specs=[
            pl.BlockSpec((NC, R, D), lambda i: (0, i, 0)),
            pl.BlockSpec((R, 1), lambda i: (i, 0)),
            pl.BlockSpec((1, D), lambda i: (0, 0)),
            pl.BlockSpec((D, D_OUT), lambda i: (0, 0)),
            pl.BlockSpec((1, D_OUT), lambda i: (0, 0)),
        ],
        out_specs=pl.BlockSpec((R, D_OUT), lambda i: (i, 0)),
        out_shape=jax.ShapeDtypeStruct((N, D_OUT), jnp.float32),
    )(p, dinv, b3, W_out, b_out)


def kernel(x, edge_index, W1, b1, W2, b2, W3, b3, W_out, b_out):
    src = edge_index[0].reshape(NW, NCHNK, K)
    dst = edge_index[1].reshape(NW, NCHNK, K)
    ones = jnp.ones((K, D), jnp.float32)
    z128 = jnp.zeros((RPT, D), jnp.float32)

    hist = _sc_deg(dst, ones, z128)
    g1, dinv = _tc_first(x, W1, hist)
    p = _sc_agg(g1, src, dst, z128)
    g2 = _tc_mid(p, dinv, b1.reshape(1, -1), W2)
    p = _sc_agg(g2, src, dst, z128)
    g3 = _tc_mid(p, dinv, b2.reshape(1, -1), W3)
    p = _sc_agg(g3, src, dst, z128)
    return _tc_final(p, dinv, b3.reshape(1, -1), W_out, b_out.reshape(1, -1))


# speedup vs baseline: 16.0760x; 1.0294x over previous
import functools

import jax
import jax.numpy as jnp
from jax import lax
from jax.experimental import pallas as pl
from jax.experimental.pallas import tpu as pltpu
from jax.experimental.pallas import tpu_sc as plsc

N = 10000
E = 320000
D = 128
D_OUT = 64

NC, NS = 2, 16
NW = NC * NS
K = 96
NCH = 107
EPW = NCH * K
E_PAD = EPW * NW
NP = 10240
RPT = NP // NS

_mesh = plsc.VectorSubcoreMesh(core_axis_name="c", subcore_axis_name="s")


@functools.partial(
    pl.kernel,
    out_type=jax.ShapeDtypeStruct((NC, NP, D), jnp.float32),
    mesh=_mesh,
    scratch_types=[
        pltpu.VMEM((NCH, K), jnp.int32),
        pltpu.VMEM((K, D), jnp.float32),
        pltpu.VMEM_SHARED((NP, D), jnp.float32),
    ],
)
def _sc_deg(dst_hbm, ones_hbm, z128_hbm, out_hbm, dst_v, ones_v, hist):
    c = lax.axis_index("c")
    s = lax.axis_index("s")
    wid = s * NC + c
    pltpu.sync_copy(dst_hbm.at[wid], dst_v)
    pltpu.sync_copy(ones_hbm, ones_v)
    pltpu.sync_copy(z128_hbm, hist.at[pl.ds(s * RPT, RPT)])
    plsc.subcore_barrier()

    def chunk(i, carry):
        pltpu.sync_copy(ones_v, hist.at[dst_v.at[i]], add=True)
        return carry

    lax.fori_loop(0, NCH, chunk, 0)
    plsc.subcore_barrier()
    pltpu.sync_copy(hist.at[pl.ds(s * RPT, RPT)],
                    out_hbm.at[c, pl.ds(s * RPT, RPT)])


@functools.partial(
    pl.kernel,
    out_type=jax.ShapeDtypeStruct((NC, NP, D), jnp.float32),
    mesh=_mesh,
    scratch_types=[
        pltpu.VMEM((2 * NCH, K), jnp.int32),
        pltpu.VMEM_SHARED((NP, D), jnp.float32),
        pltpu.VMEM((K, D), jnp.float32),
        pltpu.SemaphoreType.DMA,
    ],
)
def _sc_agg(g_hbm, idx_hbm, z128_hbm, zk_hbm, out_hbm, idx_v, acc, buf, gsem):
    c = lax.axis_index("c")
    s = lax.axis_index("s")
    wid = s * NC + c
    pltpu.sync_copy(idx_hbm.at[wid], idx_v)
    rows0 = s * RPT

    @pl.when(c == 0)
    def _init_self_loop():
        pltpu.sync_copy(g_hbm.at[pl.ds(rows0, RPT)], acc.at[pl.ds(rows0, RPT)])

    @pl.when(c == 1)
    def _init_zero():
        pltpu.sync_copy(z128_hbm, acc.at[pl.ds(rows0, RPT)])

    plsc.subcore_barrier()

    pltpu.async_copy(g_hbm.at[idx_v.at[0]], buf, gsem)

    def body(i, carry):
        pltpu.make_async_copy(zk_hbm, buf, gsem).wait()
        pltpu.sync_copy(buf, acc.at[idx_v.at[NCH + i]], add=True)
        nxt = lax.rem(i + 1, NCH)
        pltpu.async_copy(g_hbm.at[idx_v.at[nxt]], buf, gsem)
        return carry

    lax.fori_loop(0, NCH, body, 0)
    pltpu.make_async_copy(zk_hbm, buf, gsem).wait()
    plsc.subcore_barrier()
    pltpu.sync_copy(acc.at[pl.ds(rows0, RPT)],
                    out_hbm.at[c, pl.ds(rows0, RPT)])


R = 1000
GRID = N // R


def _tc_first_body(x_ref, w_ref, hist_ref, g_ref, dinv_ref):
    deg = hist_ref[0, :, 0:1] + hist_ref[1, :, 0:1] + 1.0
    dinv = lax.rsqrt(deg)
    h = jnp.dot(x_ref[...], w_ref[...], preferred_element_type=jnp.float32)
    g_ref[...] = h * dinv
    dinv_ref[...] = dinv


def _tc_first(x, W1, hist):
    return pl.pallas_call(
        _tc_first_body,
        grid=(GRID,),
        in_specs=[
            pl.BlockSpec((R, D), lambda i: (i, 0)),
            pl.BlockSpec((D, D), lambda i: (0, 0)),
            pl.BlockSpec((NC, R, D), lambda i: (0, i, 0)),
        ],
        out_specs=[
            pl.BlockSpec((R, D), lambda i: (i, 0)),
            pl.BlockSpec((R, 1), lambda i: (i, 0)),
        ],
        out_shape=[
            jax.ShapeDtypeStruct((NP, D), jnp.float32),
            jax.ShapeDtypeStruct((N, 1), jnp.float32),
        ],
    )(x, W1, hist)


def _tc_mid_body(p_ref, dinv_ref, b_ref, w_ref, g_ref):
    dinv = dinv_ref[...]
    agg = p_ref[0] + p_ref[1]
    xa = jnp.maximum(dinv * agg + b_ref[...], 0.0)
    h = jnp.dot(xa, w_ref[...], preferred_element_type=jnp.float32)
    g_ref[...] = h * dinv


def _tc_mid(p, dinv, b, W):
    return pl.pallas_call(
        _tc_mid_body,
        grid=(GRID,),
        in_specs=[
            pl.BlockSpec((NC, R, D), lambda i: (0, i, 0)),
            pl.BlockSpec((R, 1), lambda i: (i, 0)),
            pl.BlockSpec((1, D), lambda i: (0, 0)),
            pl.BlockSpec((D, D), lambda i: (0, 0)),
        ],
        out_specs=pl.BlockSpec((R, D), lambda i: (i, 0)),
        out_shape=jax.ShapeDtypeStruct((NP, D), jnp.float32),
    )(p, dinv, b, W)


def _tc_final_body(p_ref, dinv_ref, b3_ref, wout_ref, bout_ref, y_ref):
    x4 = dinv_ref[...] * (p_ref[0] + p_ref[1]) + b3_ref[...]
    y_ref[...] = (jnp.dot(x4, wout_ref[...], preferred_element_type=jnp.float32)
                  + bout_ref[...])


def _tc_final(p, dinv, b3, W_out, b_out):
    return pl.pallas_call(
        _tc_final_body,
        grid=(GRID,),
        in_specs=[
            pl.BlockSpec((NC, R, D), lambda i: (0, i, 0)),
            pl.BlockSpec((R, 1), lambda i: (i, 0)),
            pl.BlockSpec((1, D), lambda i: (0, 0)),
            pl.BlockSpec((D, D_OUT), lambda i: (0, 0)),
            pl.BlockSpec((1, D_OUT), lambda i: (0, 0)),
        ],
        out_specs=pl.BlockSpec((R, D_OUT), lambda i: (i, 0)),
        out_shape=jax.ShapeDtypeStruct((N, D_OUT), jnp.float32),
    )(p, dinv, b3, W_out, b_out)


def kernel(x, edge_index, W1, b1, W2, b2, W3, b3, W_out, b_out):
    npad = E_PAD - E
    fill = jnp.arange(npad, dtype=jnp.int32) % (NP - N)
    srcp = jnp.concatenate([edge_index[0], fill]).reshape(NW, NCH, K)
    dstp = jnp.concatenate([edge_index[1], N + fill]).reshape(NW, NCH, K)
    idx = jnp.concatenate([srcp, dstp], axis=1)
    ones = jnp.ones((K, D), jnp.float32)
    z128 = jnp.zeros((RPT, D), jnp.float32)
    zk = jnp.zeros((K, D), jnp.float32)

    hist = _sc_deg(dstp, ones, z128)
    g1, dinv = _tc_first(x, W1, hist)
    p = _sc_agg(g1, idx, z128, zk)
    g2 = _tc_mid(p, dinv, b1.reshape(1, -1), W2)
    p = _sc_agg(g2, idx, z128, zk)
    g3 = _tc_mid(p, dinv, b2.reshape(1, -1), W3)
    p = _sc_agg(g3, idx, z128, zk)
    return _tc_final(p, dinv, b3.reshape(1, -1), W_out, b_out.reshape(1, -1))


# speedup vs baseline: 16.7164x; 1.0398x over previous
import functools

import jax
import jax.numpy as jnp
from jax import lax
from jax.experimental import pallas as pl
from jax.experimental.pallas import tpu as pltpu
from jax.experimental.pallas import tpu_sc as plsc

N = 10000
E = 320000
D = 128
D_OUT = 64

NC, NS = 2, 16
NW = NC * NS
K = 112
NCH = 92
EPW = NCH * K
E_PAD = EPW * NW
NP = 10240
RPT = NP // NS

_mesh = plsc.VectorSubcoreMesh(core_axis_name="c", subcore_axis_name="s")


@functools.partial(
    pl.kernel,
    out_type=jax.ShapeDtypeStruct((NC, NP, D), jnp.float32),
    mesh=_mesh,
    scratch_types=[
        pltpu.VMEM((NCH, K), jnp.int32),
        pltpu.VMEM((K, D), jnp.float32),
        pltpu.VMEM_SHARED((NP, D), jnp.float32),
    ],
)
def _sc_deg(dst_hbm, ones_hbm, z128_hbm, out_hbm, dst_v, ones_v, hist):
    c = lax.axis_index("c")
    s = lax.axis_index("s")
    wid = s * NC + c
    pltpu.sync_copy(dst_hbm.at[wid], dst_v)
    pltpu.sync_copy(ones_hbm, ones_v)
    pltpu.sync_copy(z128_hbm, hist.at[pl.ds(s * RPT, RPT)])
    plsc.subcore_barrier()

    def chunk(i, carry):
        pltpu.sync_copy(ones_v, hist.at[dst_v.at[i]], add=True)
        return carry

    lax.fori_loop(0, NCH, chunk, 0)
    plsc.subcore_barrier()
    pltpu.sync_copy(hist.at[pl.ds(s * RPT, RPT)],
                    out_hbm.at[c, pl.ds(s * RPT, RPT)])


@functools.partial(
    pl.kernel,
    out_type=jax.ShapeDtypeStruct((NC, NP, D), jnp.float32),
    mesh=_mesh,
    scratch_types=[
        pltpu.VMEM((2 * NCH, K), jnp.int32),
        pltpu.VMEM_SHARED((NP, D), jnp.float32),
        pltpu.VMEM((K, D), jnp.float32),
        pltpu.SemaphoreType.DMA,
    ],
)
def _sc_agg(g_hbm, idx_hbm, z128_hbm, zk_hbm, out_hbm, idx_v, acc, buf, gsem):
    c = lax.axis_index("c")
    s = lax.axis_index("s")
    wid = s * NC + c
    pltpu.sync_copy(idx_hbm.at[wid], idx_v)
    rows0 = s * RPT

    @pl.when(c == 0)
    def _init_self_loop():
        pltpu.sync_copy(g_hbm.at[pl.ds(rows0, RPT)], acc.at[pl.ds(rows0, RPT)])

    @pl.when(c == 1)
    def _init_zero():
        pltpu.sync_copy(z128_hbm, acc.at[pl.ds(rows0, RPT)])

    plsc.subcore_barrier()

    pltpu.async_copy(g_hbm.at[idx_v.at[0]], buf, gsem)

    def body(i, carry):
        pltpu.make_async_copy(zk_hbm, buf, gsem).wait()
        pltpu.sync_copy(buf, acc.at[idx_v.at[NCH + i]], add=True)
        nxt = lax.rem(i + 1, NCH)
        pltpu.async_copy(g_hbm.at[idx_v.at[nxt]], buf, gsem)
        return carry

    lax.fori_loop(0, NCH, body, 0)
    pltpu.make_async_copy(zk_hbm, buf, gsem).wait()
    plsc.subcore_barrier()
    pltpu.sync_copy(acc.at[pl.ds(rows0, RPT)],
                    out_hbm.at[c, pl.ds(rows0, RPT)])


R = 1000
GRID = N // R


def _tc_first_body(x_ref, w_ref, hist_ref, g_ref, dinv_ref):
    deg = hist_ref[0, :, 0:1] + hist_ref[1, :, 0:1] + 1.0
    dinv = lax.rsqrt(deg)
    h = jnp.dot(x_ref[...], w_ref[...], preferred_element_type=jnp.float32)
    g_ref[...] = h * dinv
    dinv_ref[...] = dinv


def _tc_first(x, W1, hist):
    return pl.pallas_call(
        _tc_first_body,
        grid=(GRID,),
        in_specs=[
            pl.BlockSpec((R, D), lambda i: (i, 0)),
            pl.BlockSpec((D, D), lambda i: (0, 0)),
            pl.BlockSpec((NC, R, D), lambda i: (0, i, 0)),
        ],
        out_specs=[
            pl.BlockSpec((R, D), lambda i: (i, 0)),
            pl.BlockSpec((R, 1), lambda i: (i, 0)),
        ],
        out_shape=[
            jax.ShapeDtypeStruct((NP, D), jnp.float32),
            jax.ShapeDtypeStruct((N, 1), jnp.float32),
        ],
    )(x, W1, hist)


def _tc_mid_body(p_ref, dinv_ref, b_ref, w_ref, g_ref):
    dinv = dinv_ref[...]
    agg = p_ref[0] + p_ref[1]
    xa = jnp.maximum(dinv * agg + b_ref[...], 0.0)
    h = jnp.dot(xa, w_ref[...], preferred_element_type=jnp.float32)
    g_ref[...] = h * dinv


def _tc_mid(p, dinv, b, W):
    return pl.pallas_call(
        _tc_mid_body,
        grid=(GRID,),
        in_specs=[
            pl.BlockSpec((NC, R, D), lambda i: (0, i, 0)),
            pl.BlockSpec((R, 1), lambda i: (i, 0)),
            pl.BlockSpec((1, D), lambda i: (0, 0)),
            pl.BlockSpec((D, D), lambda i: (0, 0)),
        ],
        out_specs=pl.BlockSpec((R, D), lambda i: (i, 0)),
        out_shape=jax.ShapeDtypeStruct((NP, D), jnp.float32),
    )(p, dinv, b, W)


def _tc_final_body(p_ref, dinv_ref, b3_ref, wout_ref, bout_ref, y_ref):
    x4 = dinv_ref[...] * (p_ref[0] + p_ref[1]) + b3_ref[...]
    y_ref[...] = (jnp.dot(x4, wout_ref[...], preferred_element_type=jnp.float32)
                  + bout_ref[...])


def _tc_final(p, dinv, b3, W_out, b_out):
    return pl.pallas_call(
        _tc_final_body,
        grid=(GRID,),
        in_specs=[
            pl.BlockSpec((NC, R, D), lambda i: (0, i, 0)),
            pl.BlockSpec((R, 1), lambda i: (i, 0)),
            pl.BlockSpec((1, D), lambda i: (0, 0)),
            pl.BlockSpec((D, D_OUT), lambda i: (0, 0)),
            pl.BlockSpec((1, D_OUT), lambda i: (0, 0)),
        ],
        out_specs=pl.BlockSpec((R, D_OUT), lambda i: (i, 0)),
        out_shape=jax.ShapeDtypeStruct((N, D_OUT), jnp.float32),
    )(p, dinv, b3, W_out, b_out)


def kernel(x, edge_index, W1, b1, W2, b2, W3, b3, W_out, b_out):
    npad = E_PAD - E
    fill = jnp.arange(npad, dtype=jnp.int32) % (NP - N)
    srcp = jnp.concatenate([edge_index[0], fill]).reshape(NW, NCH, K)
    dstp = jnp.concatenate([edge_index[1], N + fill]).reshape(NW, NCH, K)
    idx = jnp.concatenate([srcp, dstp], axis=1)
    ones = jnp.ones((K, D), jnp.float32)
    z128 = jnp.zeros((RPT, D), jnp.float32)
    zk = jnp.zeros((K, D), jnp.float32)

    hist = _sc_deg(dstp, ones, z128)
    g1, dinv = _tc_first(x, W1, hist)
    p = _sc_agg(g1, idx, z128, zk)
    g2 = _tc_mid(p, dinv, b1.reshape(1, -1), W2)
    p = _sc_agg(g2, idx, z128, zk)
    g3 = _tc_mid(p, dinv, b2.reshape(1, -1), W3)
    p = _sc_agg(g3, idx, z128, zk)
    return _tc_final(p, dinv, b3.reshape(1, -1), W_out, b_out.reshape(1, -1))


# speedup vs baseline: 18.3005x; 1.0948x over previous
import functools

import jax
import jax.numpy as jnp
from jax import lax
from jax.experimental import pallas as pl
from jax.experimental.pallas import tpu as pltpu
from jax.experimental.pallas import tpu_sc as plsc

N = 10000
E = 320000
D = 128
D_OUT = 64

NC, NS = 2, 16
NW = NC * NS
K = 112
NCH = 92
EPW = NCH * K
E_PAD = EPW * NW
NP = 10240
RPT = NP // NS

_mesh = plsc.VectorSubcoreMesh(core_axis_name="c", subcore_axis_name="s")


NPR = NP // D
VPC = K // 16


@functools.partial(
    pl.kernel,
    out_type=jax.ShapeDtypeStruct((NW, 1, NP), jnp.float32),
    mesh=_mesh,
    scratch_types=[
        pltpu.VMEM((NCH, K), jnp.int32),
        pltpu.VMEM((NP,), jnp.float32),
    ],
    compiler_params=pltpu.CompilerParams(needs_layout_passes=False),
)
def _sc_deg(dst_hbm, out_hbm, dst_v, hloc):
    c = lax.axis_index("c")
    s = lax.axis_index("s")
    wid = s * NC + c
    pltpu.sync_copy(dst_hbm.at[wid], dst_v)

    zero16 = jnp.zeros((16,), jnp.float32)

    def zvec(r, carry):
        hloc[pl.ds(r * 16, 16)] = zero16
        return carry

    lax.fori_loop(0, NP // 16, zvec, 0)

    ones16 = jnp.ones((16,), jnp.float32)

    def chunk(i, carry):
        def vec(j, carry2):
            idx = dst_v[i, pl.ds(j * 16, 16)]
            plsc.addupdate_scatter(hloc, [idx], ones16)
            return carry2
        return lax.fori_loop(0, VPC, vec, carry)

    lax.fori_loop(0, NCH, chunk, 0)
    pltpu.sync_copy(hloc, out_hbm.at[wid, 0])


@functools.partial(
    pl.kernel,
    out_type=jax.ShapeDtypeStruct((NC, NP, D), jnp.float32),
    mesh=_mesh,
    scratch_types=[
        pltpu.VMEM((2 * NCH, K), jnp.int32),
        pltpu.VMEM_SHARED((NP, D), jnp.float32),
        pltpu.VMEM((K, D), jnp.float32),
        pltpu.SemaphoreType.DMA,
    ],
)
def _sc_agg(g_hbm, idx_hbm, z128_hbm, zk_hbm, out_hbm, idx_v, acc, buf, gsem):
    c = lax.axis_index("c")
    s = lax.axis_index("s")
    wid = s * NC + c
    pltpu.sync_copy(idx_hbm.at[wid], idx_v)
    rows0 = s * RPT

    @pl.when(c == 0)
    def _init_self_loop():
        pltpu.sync_copy(g_hbm.at[pl.ds(rows0, RPT)], acc.at[pl.ds(rows0, RPT)])

    @pl.when(c == 1)
    def _init_zero():
        pltpu.sync_copy(z128_hbm, acc.at[pl.ds(rows0, RPT)])

    plsc.subcore_barrier()

    pltpu.async_copy(g_hbm.at[idx_v.at[0]], buf, gsem)

    def body(i, carry):
        pltpu.make_async_copy(zk_hbm, buf, gsem).wait()
        pltpu.sync_copy(buf, acc.at[idx_v.at[NCH + i]], add=True)
        nxt = lax.rem(i + 1, NCH)
        pltpu.async_copy(g_hbm.at[idx_v.at[nxt]], buf, gsem)
        return carry

    lax.fori_loop(0, NCH, body, 0)
    pltpu.make_async_copy(zk_hbm, buf, gsem).wait()
    plsc.subcore_barrier()
    pltpu.sync_copy(acc.at[pl.ds(rows0, RPT)],
                    out_hbm.at[c, pl.ds(rows0, RPT)])


R = 1000
GRID = N // R


def _tc_prep_body(hist_ref, dinv_ref):
    acc = hist_ref[0] + 1.0
    for w in range(1, NW):
        acc = acc + hist_ref[w]
    dinv_ref[...] = lax.rsqrt(acc)


def _tc_prep(hist):
    return pl.pallas_call(
        _tc_prep_body,
        grid=(1,),
        in_specs=[pl.BlockSpec((NW, 1, NP), lambda i: (0, 0, 0))],
        out_specs=pl.BlockSpec((1, NP), lambda i: (0, 0)),
        out_shape=jax.ShapeDtypeStruct((1, NP), jnp.float32),
    )(hist)


def _tc_first_body(x_ref, w_ref, dinv_ref, g_ref):
    h = jnp.dot(x_ref[...], w_ref[...], preferred_element_type=jnp.float32)
    g_ref[...] = h * dinv_ref[...]


def _tc_first(x, W1, dinv):
    return pl.pallas_call(
        _tc_first_body,
        grid=(GRID,),
        in_specs=[
            pl.BlockSpec((R, D), lambda i: (i, 0)),
            pl.BlockSpec((D, D), lambda i: (0, 0)),
            pl.BlockSpec((R, 1), lambda i: (i, 0)),
        ],
        out_specs=pl.BlockSpec((R, D), lambda i: (i, 0)),
        out_shape=jax.ShapeDtypeStruct((NP, D), jnp.float32),
    )(x, W1, dinv)


def _tc_mid_body(p_ref, dinv_ref, b_ref, w_ref, g_ref):
    dinv = dinv_ref[...]
    agg = p_ref[0] + p_ref[1]
    xa = jnp.maximum(dinv * agg + b_ref[...], 0.0)
    h = jnp.dot(xa, w_ref[...], preferred_element_type=jnp.float32)
    g_ref[...] = h * dinv


def _tc_mid(p, dinv, b, W):
    return pl.pallas_call(
        _tc_mid_body,
        grid=(GRID,),
        in_specs=[
            pl.BlockSpec((NC, R, D), lambda i: (0, i, 0)),
            pl.BlockSpec((R, 1), lambda i: (i, 0)),
            pl.BlockSpec((1, D), lambda i: (0, 0)),
            pl.BlockSpec((D, D), lambda i: (0, 0)),
        ],
        out_specs=pl.BlockSpec((R, D), lambda i: (i, 0)),
        out_shape=jax.ShapeDtypeStruct((NP, D), jnp.float32),
    )(p, dinv, b, W)


def _tc_final_body(p_ref, dinv_ref, b3_ref, wout_ref, bout_ref, y_ref):
    x4 = dinv_ref[...] * (p_ref[0] + p_ref[1]) + b3_ref[...]
    y_ref[...] = (jnp.dot(x4, wout_ref[...], preferred_element_type=jnp.float32)
                  + bout_ref[...])


def _tc_final(p, dinv, b3, W_out, b_out):
    return pl.pallas_call(
        _tc_final_body,
        grid=(GRID,),
        in_specs=[
            pl.BlockSpec((NC, R, D), lambda i: (0, i, 0)),
            pl.BlockSpec((R, 1), lambda i: (i, 0)),
            pl.BlockSpec((1, D), lambda i: (0, 0)),
            pl.BlockSpec((D, D_OUT), lambda i: (0, 0)),
            pl.BlockSpec((1, D_OUT), lambda i: (0, 0)),
        ],
        out_specs=pl.BlockSpec((R, D_OUT), lambda i: (i, 0)),
        out_shape=jax.ShapeDtypeStruct((N, D_OUT), jnp.float32),
    )(p, dinv, b3, W_out, b_out)


def kernel(x, edge_index, W1, b1, W2, b2, W3, b3, W_out, b_out):
    npad = E_PAD - E
    fill = jnp.arange(npad, dtype=jnp.int32) % (NP - N)
    srcp = jnp.concatenate([edge_index[0], fill]).reshape(NW, NCH, K)
    dstp = jnp.concatenate([edge_index[1], N + fill]).reshape(NW, NCH, K)
    idx = jnp.concatenate([srcp, dstp], axis=1)
    z128 = jnp.zeros((RPT, D), jnp.float32)
    zk = jnp.zeros((K, D), jnp.float32)

    hist = _sc_deg(dstp)
    dinv = _tc_prep(hist).reshape(NP)[:N].reshape(N, 1)
    g1 = _tc_first(x, W1, dinv)
    p = _sc_agg(g1, idx, z128, zk)
    g2 = _tc_mid(p, dinv, b1.reshape(1, -1), W2)
    p = _sc_agg(g2, idx, z128, zk)
    g3 = _tc_mid(p, dinv, b2.reshape(1, -1), W3)
    p = _sc_agg(g3, idx, z128, zk)
    return _tc_final(p, dinv, b3.reshape(1, -1), W_out, b_out.reshape(1, -1))


# speedup vs baseline: 18.9470x; 1.0353x over previous
import functools

import jax
import jax.numpy as jnp
from jax import lax
from jax.experimental import pallas as pl
from jax.experimental.pallas import tpu as pltpu
from jax.experimental.pallas import tpu_sc as plsc

N = 10000
E = 320000
D = 128
D_OUT = 64

NC, NS = 2, 16
NW = NC * NS
K = 128
NCH = 81
EPW = NCH * K
E_PAD = EPW * NW
NP = 10240
NPA = 10112
RPT = NPA // NS

_mesh = plsc.VectorSubcoreMesh(core_axis_name="c", subcore_axis_name="s")


NPR = NP // D
VPC = K // 16


@functools.partial(
    pl.kernel,
    out_type=jax.ShapeDtypeStruct((NW, 1, NP), jnp.float32),
    mesh=_mesh,
    scratch_types=[
        pltpu.VMEM((NCH, K), jnp.int32),
        pltpu.VMEM((NP,), jnp.float32),
    ],
    compiler_params=pltpu.CompilerParams(needs_layout_passes=False),
)
def _sc_deg(dst_hbm, out_hbm, dst_v, hloc):
    c = lax.axis_index("c")
    s = lax.axis_index("s")
    wid = s * NC + c
    pltpu.sync_copy(dst_hbm.at[wid], dst_v)

    zero16 = jnp.zeros((16,), jnp.float32)

    def zvec(r, carry):
        hloc[pl.ds(r * 16, 16)] = zero16
        return carry

    lax.fori_loop(0, NP // 16, zvec, 0)

    ones16 = jnp.ones((16,), jnp.float32)

    def chunk(i, carry):
        def vec(j, carry2):
            idx = dst_v[i, pl.ds(j * 16, 16)]
            plsc.addupdate_scatter(hloc, [idx], ones16)
            return carry2
        return lax.fori_loop(0, VPC, vec, carry)

    lax.fori_loop(0, NCH, chunk, 0)
    pltpu.sync_copy(hloc, out_hbm.at[wid, 0])


@functools.partial(
    pl.kernel,
    out_type=jax.ShapeDtypeStruct((NC, NPA, D), jnp.float32),
    mesh=_mesh,
    scratch_types=[
        pltpu.VMEM((2 * NCH, K), jnp.int32),
        pltpu.VMEM_SHARED((NPA, D), jnp.float32),
        pltpu.VMEM((K, D), jnp.float32),
        pltpu.SemaphoreType.DMA,
    ],
)
def _sc_agg(g_hbm, idx_hbm, z128_hbm, zk_hbm, out_hbm, idx_v, acc, buf, gsem):
    c = lax.axis_index("c")
    s = lax.axis_index("s")
    wid = s * NC + c
    pltpu.sync_copy(idx_hbm.at[wid], idx_v)
    rows0 = s * RPT

    @pl.when(c == 0)
    def _init_self_loop():
        pltpu.sync_copy(g_hbm.at[pl.ds(rows0, RPT)], acc.at[pl.ds(rows0, RPT)])

    @pl.when(c == 1)
    def _init_zero():
        pltpu.sync_copy(z128_hbm, acc.at[pl.ds(rows0, RPT)])

    plsc.subcore_barrier()

    pltpu.async_copy(g_hbm.at[idx_v.at[0]], buf, gsem)

    def body(i, carry):
        pltpu.make_async_copy(zk_hbm, buf, gsem).wait()
        pltpu.sync_copy(buf, acc.at[idx_v.at[NCH + i]], add=True)
        nxt = lax.rem(i + 1, NCH)
        pltpu.async_copy(g_hbm.at[idx_v.at[nxt]], buf, gsem)
        return carry

    lax.fori_loop(0, NCH, body, 0)
    pltpu.make_async_copy(zk_hbm, buf, gsem).wait()
    plsc.subcore_barrier()
    pltpu.sync_copy(acc.at[pl.ds(rows0, RPT)],
                    out_hbm.at[c, pl.ds(rows0, RPT)])


R = 1000
GRID = N // R


def _tc_prep_body(hist_ref, dinv_ref):
    acc = hist_ref[0] + 1.0
    for w in range(1, NW):
        acc = acc + hist_ref[w]
    dinv_ref[...] = lax.rsqrt(acc)


def _tc_prep(hist):
    return pl.pallas_call(
        _tc_prep_body,
        grid=(1,),
        in_specs=[pl.BlockSpec((NW, 1, NP), lambda i: (0, 0, 0))],
        out_specs=pl.BlockSpec((1, NP), lambda i: (0, 0)),
        out_shape=jax.ShapeDtypeStruct((1, NP), jnp.float32),
    )(hist)


def _tc_first_body(x_ref, w_ref, dinv_ref, g_ref):
    h = jnp.dot(x_ref[...], w_ref[...], preferred_element_type=jnp.float32)
    g_ref[...] = h * dinv_ref[...]


def _tc_first(x, W1, dinv):
    return pl.pallas_call(
        _tc_first_body,
        grid=(GRID,),
        in_specs=[
            pl.BlockSpec((R, D), lambda i: (i, 0)),
            pl.BlockSpec((D, D), lambda i: (0, 0)),
            pl.BlockSpec((R, 1), lambda i: (i, 0)),
        ],
        out_specs=pl.BlockSpec((R, D), lambda i: (i, 0)),
        out_shape=jax.ShapeDtypeStruct((NP, D), jnp.float32),
    )(x, W1, dinv)


def _tc_mid_body(p_ref, dinv_ref, b_ref, w_ref, g_ref):
    dinv = dinv_ref[...]
    agg = p_ref[0] + p_ref[1]
    xa = jnp.maximum(dinv * agg + b_ref[...], 0.0)
    h = jnp.dot(xa, w_ref[...], preferred_element_type=jnp.float32)
    g_ref[...] = h * dinv


def _tc_mid(p, dinv, b, W):
    return pl.pallas_call(
        _tc_mid_body,
        grid=(GRID,),
        in_specs=[
            pl.BlockSpec((NC, R, D), lambda i: (0, i, 0)),
            pl.BlockSpec((R, 1), lambda i: (i, 0)),
            pl.BlockSpec((1, D), lambda i: (0, 0)),
            pl.BlockSpec((D, D), lambda i: (0, 0)),
        ],
        out_specs=pl.BlockSpec((R, D), lambda i: (i, 0)),
        out_shape=jax.ShapeDtypeStruct((NP, D), jnp.float32),
    )(p, dinv, b, W)


def _tc_final_body(p_ref, dinv_ref, b3_ref, wout_ref, bout_ref, y_ref):
    x4 = dinv_ref[...] * (p_ref[0] + p_ref[1]) + b3_ref[...]
    y_ref[...] = (jnp.dot(x4, wout_ref[...], preferred_element_type=jnp.float32)
                  + bout_ref[...])


def _tc_final(p, dinv, b3, W_out, b_out):
    return pl.pallas_call(
        _tc_final_body,
        grid=(GRID,),
        in_specs=[
            pl.BlockSpec((NC, R, D), lambda i: (0, i, 0)),
            pl.BlockSpec((R, 1), lambda i: (i, 0)),
            pl.BlockSpec((1, D), lambda i: (0, 0)),
            pl.BlockSpec((D, D_OUT), lambda i: (0, 0)),
            pl.BlockSpec((1, D_OUT), lambda i: (0, 0)),
        ],
        out_specs=pl.BlockSpec((R, D_OUT), lambda i: (i, 0)),
        out_shape=jax.ShapeDtypeStruct((N, D_OUT), jnp.float32),
    )(p, dinv, b3, W_out, b_out)


def kernel(x, edge_index, W1, b1, W2, b2, W3, b3, W_out, b_out):
    npad = E_PAD - E
    fill = jnp.arange(npad, dtype=jnp.int32) % (NPA - N)
    srcp = jnp.concatenate([edge_index[0], fill]).reshape(NW, NCH, K)
    dstp = jnp.concatenate([edge_index[1], N + fill]).reshape(NW, NCH, K)
    idx = jnp.concatenate([srcp, dstp], axis=1)
    z128 = jnp.zeros((RPT, D), jnp.float32)
    zk = jnp.zeros((K, D), jnp.float32)

    hist = _sc_deg(dstp)
    dinv = _tc_prep(hist).reshape(NP)[:N].reshape(N, 1)
    g1 = _tc_first(x, W1, dinv)
    p = _sc_agg(g1, idx, z128, zk)
    g2 = _tc_mid(p, dinv, b1.reshape(1, -1), W2)
    p = _sc_agg(g2, idx, z128, zk)
    g3 = _tc_mid(p, dinv, b2.reshape(1, -1), W3)
    p = _sc_agg(g3, idx, z128, zk)
    return _tc_final(p, dinv, b3.reshape(1, -1), W_out, b_out.reshape(1, -1))
